# baseline (speedup 1.0000x reference)
import jax
import jax.numpy as jnp
from jax import lax
from jax.experimental import pallas as pl
from jax.experimental.pallas import tpu as pltpu
from jax.experimental.pallas import tpu_sc as plsc

VOCAB = 1000000
D_MODEL = 128
BATCH = 4
SEQ_LEN = 8192

NC = 2
NS = 16
NW = NC * NS

S_PER_W = SEQ_LEN // NW
CH = 64
HALVES = S_PER_W // CH
NCH = BATCH * HALVES
NBUF = 8


def _body(idx_hbm, tok_hbm, pos_hbm, out_hbm, *rest):
    idx_bufs = list(rest[:NCH])
    pos_sh, acc_v, sem_i, sem_p, sem_c, sem_g, sem_o = rest[NCH:]
    wid = lax.axis_index("s") * NC + lax.axis_index("c")
    sid = lax.axis_index("s")
    s0 = wid * S_PER_W

    def bh(t):
        return t // HALVES, (t % HALVES) * CH

    my_pos = pos_sh.at[pl.ds(sid * S_PER_W, S_PER_W)]
    pltpu.async_copy(pos_hbm.at[pl.ds(s0, S_PER_W)], my_pos, sem_p)
    for t in range(NCH):
        b, h = bh(t)
        pltpu.async_copy(idx_hbm.at[b, pl.ds(s0 + h, CH)], idx_bufs[t], sem_i)
    def spos(t):
        return pos_sh.at[pl.ds(sid * S_PER_W + bh(t)[1], CH)]

    def oslice(t):
        b, h = bh(t)
        return out_hbm.at[b, pl.ds(s0 + h, CH)]

    def issue_posc(t):
        pltpu.async_copy(spos(t), acc_v.at[t % NBUF], sem_c.at[t % NBUF])

    def wait_posc(t):
        pltpu.make_async_copy(spos(t), acc_v.at[t % NBUF],
                              sem_c.at[t % NBUF]).wait()

    def issue_gather(t):
        pltpu.async_copy(tok_hbm.at[idx_bufs[t]], acc_v.at[t % NBUF],
                         sem_g.at[t % NBUF], add=True)

    def wait_gather(t):
        pltpu.make_async_copy(tok_hbm.at[idx_bufs[t]], acc_v.at[t % NBUF],
                              sem_g.at[t % NBUF]).wait()

    def issue_out(t):
        pltpu.async_copy(acc_v.at[t % NBUF], oslice(t), sem_o.at[t % NBUF])

    def wait_out(t):
        pltpu.make_async_copy(acc_v.at[t % NBUF], oslice(t),
                              sem_o.at[t % NBUF]).wait()

    pltpu.make_async_copy(pos_hbm.at[pl.ds(s0, S_PER_W)], my_pos, sem_p).wait()
    for t in range(NBUF):
        issue_posc(t)
    for t in range(NCH):
        b, h = bh(t)
        pltpu.make_async_copy(idx_hbm.at[b, pl.ds(s0 + h, CH)],
                              idx_bufs[t], sem_i).wait()
    for t in range(NCH):
        wait_posc(t)
        issue_gather(t)
        if t >= 1:
            wait_gather(t - 1)
            issue_out(t - 1)
        nxt = t + 1
        if NBUF <= nxt < NCH:
            wait_out(nxt - NBUF)
            issue_posc(nxt)
    wait_gather(NCH - 1)
    issue_out(NCH - 1)
    for t in range(NCH - NBUF, NCH):
        wait_out(t)


@jax.jit
def _embed(x, token_table, pos_table):
    mesh = plsc.VectorSubcoreMesh(core_axis_name="c", subcore_axis_name="s")
    k = pl.kernel(
        _body,
        out_type=jax.ShapeDtypeStruct((BATCH, SEQ_LEN, D_MODEL), jnp.float32),
        mesh=mesh,
        scratch_types=(
            [pltpu.VMEM((CH,), jnp.int32) for _ in range(NCH)]
            + [
                pltpu.VMEM_SHARED((NS * S_PER_W, D_MODEL), jnp.float32),
                pltpu.VMEM((NBUF, CH, D_MODEL), jnp.float32),
                pltpu.SemaphoreType.DMA,
                pltpu.SemaphoreType.DMA,
                pltpu.SemaphoreType.DMA((NBUF,)),
                pltpu.SemaphoreType.DMA((NBUF,)),
                pltpu.SemaphoreType.DMA((NBUF,)),
            ]
        ),
    )
    return k(x, token_table, pos_table)


def kernel(x, token_table, pos_table):
    return _embed(x.astype(jnp.int32), token_table, pos_table)


# speedup vs baseline: 1.0692x; 1.0692x over previous
import jax
import jax.numpy as jnp
from jax import lax
from jax.experimental import pallas as pl
from jax.experimental.pallas import tpu as pltpu
from jax.experimental.pallas import tpu_sc as plsc

VOCAB = 1000000
D_MODEL = 128
BATCH = 4
SEQ_LEN = 8192

NC = 2
NS = 16
NW = NC * NS

S_PER_W = SEQ_LEN // NW
CH = 128
HALVES = S_PER_W // CH
NCH = BATCH * HALVES
NBUF = 5


def _body(idx_hbm, tok_hbm, pos_hbm, out_hbm,
          i0, i1, i2, i3, i4, i5, i6, i7, pos_sh, acc_v,
          sem_i, sem_p, sem_c, sem_g, sem_o):
    idx_bufs = [i0, i1, i2, i3, i4, i5, i6, i7]
    wid = lax.axis_index("s") * NC + lax.axis_index("c")
    sid = lax.axis_index("s")
    s0 = wid * S_PER_W

    def bh(t):
        return t // HALVES, (t % HALVES) * CH

    my_pos = pos_sh.at[pl.ds(sid * S_PER_W, S_PER_W)]
    pltpu.async_copy(pos_hbm.at[pl.ds(s0, S_PER_W)], my_pos, sem_p)
    for t in range(NCH):
        b, h = bh(t)
        pltpu.async_copy(idx_hbm.at[b, pl.ds(s0 + h, CH)], idx_bufs[t], sem_i)
    def spos(t):
        return pos_sh.at[pl.ds(sid * S_PER_W + bh(t)[1], CH)]

    def oslice(t):
        b, h = bh(t)
        return out_hbm.at[b, pl.ds(s0 + h, CH)]

    def issue_posc(t):
        pltpu.async_copy(spos(t), acc_v.at[t % NBUF], sem_c.at[t % NBUF])

    def wait_posc(t):
        pltpu.make_async_copy(spos(t), acc_v.at[t % NBUF],
                              sem_c.at[t % NBUF]).wait()

    def issue_gather(t):
        pltpu.async_copy(tok_hbm.at[idx_bufs[t]], acc_v.at[t % NBUF],
                         sem_g.at[t % NBUF], add=True)

    def wait_gather(t):
        pltpu.make_async_copy(tok_hbm.at[idx_bufs[t]], acc_v.at[t % NBUF],
                              sem_g.at[t % NBUF]).wait()

    def issue_out(t):
        pltpu.async_copy(acc_v.at[t % NBUF], oslice(t), sem_o.at[t % NBUF])

    def wait_out(t):
        pltpu.make_async_copy(acc_v.at[t % NBUF], oslice(t),
                              sem_o.at[t % NBUF]).wait()

    pltpu.make_async_copy(pos_hbm.at[pl.ds(s0, S_PER_W)], my_pos, sem_p).wait()
    for t in range(NBUF):
        issue_posc(t)
    for t in range(NCH):
        b, h = bh(t)
        pltpu.make_async_copy(idx_hbm.at[b, pl.ds(s0 + h, CH)],
                              idx_bufs[t], sem_i).wait()
    for t in range(NCH):
        wait_posc(t)
        issue_gather(t)
        if t >= 1:
            wait_gather(t - 1)
            issue_out(t - 1)
        nxt = t + 1
        if NBUF <= nxt < NCH:
            wait_out(nxt - NBUF)
            issue_posc(nxt)
    wait_gather(NCH - 1)
    issue_out(NCH - 1)
    for t in range(NCH - NBUF, NCH):
        wait_out(t)


@jax.jit
def _embed(x, token_table, pos_table):
    mesh = plsc.VectorSubcoreMesh(core_axis_name="c", subcore_axis_name="s")
    k = pl.kernel(
        _body,
        out_type=jax.ShapeDtypeStruct((BATCH, SEQ_LEN, D_MODEL), jnp.float32),
        mesh=mesh,
        scratch_types=(
            [pltpu.VMEM((CH,), jnp.int32) for _ in range(NCH)]
            + [
                pltpu.VMEM_SHARED((NS * S_PER_W, D_MODEL), jnp.float32),
                pltpu.VMEM((NBUF, CH, D_MODEL), jnp.float32),
                pltpu.SemaphoreType.DMA,
                pltpu.SemaphoreType.DMA,
                pltpu.SemaphoreType.DMA((NBUF,)),
                pltpu.SemaphoreType.DMA((NBUF,)),
                pltpu.SemaphoreType.DMA((NBUF,)),
            ]
        ),
    )
    return k(x, token_table, pos_table)


def kernel(x, token_table, pos_table):
    return _embed(x.astype(jnp.int32), token_table, pos_table)


# speedup vs baseline: 1.0788x; 1.0090x over previous
import jax
import jax.numpy as jnp
from jax import lax
from jax.experimental import pallas as pl
from jax.experimental.pallas import tpu as pltpu
from jax.experimental.pallas import tpu_sc as plsc

VOCAB = 1000000
D_MODEL = 128
BATCH = 4
SEQ_LEN = 8192

NC = 2
NS = 16
NW = NC * NS

S_PER_W = SEQ_LEN // NW
CH = 128
HALVES = S_PER_W // CH
NCH = BATCH * HALVES
NBUF = 4


def _body(idx_hbm, tok_hbm, pos_hbm, out_hbm,
          i0, i1, i2, i3, i4, i5, i6, i7, pos_sh, acc_v,
          sem_i, sem_p, sem_c, sem_g, sem_o):
    idx_bufs = [i0, i1, i2, i3, i4, i5, i6, i7]
    wid = lax.axis_index("s") * NC + lax.axis_index("c")
    sid = lax.axis_index("s")
    s0 = wid * S_PER_W

    def bh(t):
        return t // HALVES, (t % HALVES) * CH

    my_pos = pos_sh.at[pl.ds(sid * S_PER_W, S_PER_W)]
    pltpu.async_copy(pos_hbm.at[pl.ds(s0, S_PER_W)], my_pos, sem_p)
    for t in range(NCH):
        b, h = bh(t)
        pltpu.async_copy(idx_hbm.at[b, pl.ds(s0 + h, CH)], idx_bufs[t], sem_i)
    def spos(t):
        return pos_sh.at[pl.ds(sid * S_PER_W + bh(t)[1], CH)]

    def oslice(t):
        b, h = bh(t)
        return out_hbm.at[b, pl.ds(s0 + h, CH)]

    def issue_posc(t):
        pltpu.async_copy(spos(t), acc_v.at[t % NBUF], sem_c.at[t % NBUF])

    def wait_posc(t):
        pltpu.make_async_copy(spos(t), acc_v.at[t % NBUF],
                              sem_c.at[t % NBUF]).wait()

    def issue_gather(t):
        pltpu.async_copy(tok_hbm.at[idx_bufs[t]], acc_v.at[t % NBUF],
                         sem_g.at[t % NBUF], add=True)

    def wait_gather(t):
        pltpu.make_async_copy(tok_hbm.at[idx_bufs[t]], acc_v.at[t % NBUF],
                              sem_g.at[t % NBUF]).wait()

    def issue_out(t):
        pltpu.async_copy(acc_v.at[t % NBUF], oslice(t), sem_o.at[t % NBUF])

    def wait_out(t):
        pltpu.make_async_copy(acc_v.at[t % NBUF], oslice(t),
                              sem_o.at[t % NBUF]).wait()

    pltpu.make_async_copy(pos_hbm.at[pl.ds(s0, S_PER_W)], my_pos, sem_p).wait()
    for t in range(NBUF):
        issue_posc(t)
    for t in range(NCH):
        b, h = bh(t)
        pltpu.make_async_copy(idx_hbm.at[b, pl.ds(s0 + h, CH)],
                              idx_bufs[t], sem_i).wait()
    for t in range(NCH):
        wait_posc(t)
        issue_gather(t)
        if t >= 1:
            wait_gather(t - 1)
            issue_out(t - 1)
        nxt = t + 1
        if NBUF <= nxt < NCH:
            wait_out(nxt - NBUF)
            issue_posc(nxt)
    wait_gather(NCH - 1)
    issue_out(NCH - 1)
    for t in range(NCH - NBUF, NCH):
        wait_out(t)


@jax.jit
def _embed(x, token_table, pos_table):
    mesh = plsc.VectorSubcoreMesh(core_axis_name="c", subcore_axis_name="s")
    k = pl.kernel(
        _body,
        out_type=jax.ShapeDtypeStruct((BATCH, SEQ_LEN, D_MODEL), jnp.float32),
        mesh=mesh,
        scratch_types=(
            [pltpu.VMEM((CH,), jnp.int32) for _ in range(NCH)]
            + [
                pltpu.VMEM_SHARED((NS * S_PER_W, D_MODEL), jnp.float32),
                pltpu.VMEM((NBUF, CH, D_MODEL), jnp.float32),
                pltpu.SemaphoreType.DMA,
                pltpu.SemaphoreType.DMA,
                pltpu.SemaphoreType.DMA((NBUF,)),
                pltpu.SemaphoreType.DMA((NBUF,)),
                pltpu.SemaphoreType.DMA((NBUF,)),
            ]
        ),
    )
    return k(x, token_table, pos_table)


def kernel(x, token_table, pos_table):
    return _embed(x.astype(jnp.int32), token_table, pos_table)
